# baseline (speedup 1.0000x reference)
import functools

import jax
import jax.numpy as jnp
from jax import lax
from jax.experimental import pallas as pl
from jax.experimental.pallas import tpu as pltpu
from jax.experimental.pallas import tpu_sc as plsc

N = 10000
E = 320000
D = 128
NC = 2
NS = 16
NW = NC * NS
EPW = E // NW
C = 80
NCHUNK = EPW // C
RPT = 624
REM = N - NS * RPT
ZR = 104
LANES = 16


def _lane_bcast(wv, i):
    return lax.gather(
        wv, jnp.full((LANES, 1), i, jnp.int32),
        lax.GatherDimensionNumbers(
            offset_dims=(), collapsed_slice_dims=(0,), start_index_map=(0,)),
        (1,), mode=lax.GatherScatterMode.PROMISE_IN_BOUNDS)


def _sc_body(src_h, dst_h, w_h, v_h, out_h,
             acc, src_v, dst_v, w_v, rows_v, zbuf, sem):
    c = lax.axis_index("c")
    s = lax.axis_index("s")
    wid = c * NS + s

    def zfill(i, carry):
        r = i // (D // LANES)
        k = (i % (D // LANES)) * LANES
        zbuf[r, pl.ds(k, LANES)] = jnp.zeros((LANES,), jnp.float32)
        return carry

    lax.fori_loop(0, ZR * (D // LANES), zfill, 0)
    row0 = s * RPT
    for b in range(RPT // ZR):
        pltpu.sync_copy(zbuf, acc.at[pl.ds(row0 + b * ZR, ZR)])

    @pl.when(s == NS - 1)
    def _zero_rem():
        pltpu.sync_copy(zbuf.at[pl.ds(0, REM)], acc.at[pl.ds(NS * RPT, REM)])

    plsc.subcore_barrier()

    def chunk(j, carry):
        off = wid * EPW + j * C
        pltpu.sync_copy(src_h.at[pl.ds(off, C)], src_v)
        pltpu.sync_copy(dst_h.at[pl.ds(off, C)], dst_v)
        pltpu.sync_copy(w_h.at[pl.ds(off, C)], w_v)
        pltpu.async_copy(v_h.at[src_v], rows_v, sem).wait()

        def scale(g16, carry2):
            wv = w_v[pl.ds(g16 * LANES, LANES)]
            for i in range(LANES):
                wb = _lane_bcast(wv, i)
                e = g16 * LANES + i
                for q in range(D // LANES):
                    sl = pl.ds(q * LANES, LANES)
                    rows_v[e, sl] = rows_v[e, sl] * wb
            return carry2

        lax.fori_loop(0, C // LANES, scale, 0)
        pltpu.sync_copy(rows_v, acc.at[dst_v], add=True)
        return carry

    lax.fori_loop(0, NCHUNK, chunk, 0)

    plsc.subcore_barrier()
    pltpu.sync_copy(acc.at[pl.ds(row0, RPT)], out_h.at[c, pl.ds(row0, RPT)])

    @pl.when(s == NS - 1)
    def _pub_rem():
        pltpu.sync_copy(acc.at[pl.ds(NS * RPT, REM)],
                        out_h.at[c, pl.ds(NS * RPT, REM)])


_sc_segment = pl.kernel(
    _sc_body,
    out_type=jax.ShapeDtypeStruct((NC, N, D), jnp.float32),
    mesh=plsc.VectorSubcoreMesh(core_axis_name="c", subcore_axis_name="s"),
    scratch_types=[
        pltpu.VMEM_SHARED((N, D), jnp.float32),
        pltpu.VMEM((C,), jnp.int32),
        pltpu.VMEM((C,), jnp.int32),
        pltpu.VMEM((C,), jnp.float32),
        pltpu.VMEM((C, D), jnp.float32),
        pltpu.VMEM((ZR, D), jnp.float32),
        pltpu.SemaphoreType.DMA,
    ],
)


def _mm_body(p_ref, w_ref, o_ref):
    x = p_ref[0] + p_ref[1]
    y = lax.dot_general(x, w_ref[...], (((1,), (1,)), ((), ())),
                        preferred_element_type=jnp.float32,
                        precision=lax.Precision.HIGHEST)
    o_ref[...] = jnp.maximum(y, 0.0)


_MM_BM = 1000


def _tc_linear_relu(partials, W):
    return pl.pallas_call(
        _mm_body,
        grid=(N // _MM_BM,),
        in_specs=[
            pl.BlockSpec((NC, _MM_BM, D), lambda i: (0, i, 0)),
            pl.BlockSpec((D, D), lambda i: (0, 0)),
        ],
        out_specs=pl.BlockSpec((_MM_BM, D), lambda i: (i, 0)),
        out_shape=jax.ShapeDtypeStruct((N, D), jnp.float32),
    )(partials, W)


@jax.jit
def kernel(v, edge_index, edge_weight, W):
    src = edge_index[1]
    dst = edge_index[0]
    partials = _sc_segment(src, dst, edge_weight, v)
    return _tc_linear_relu(partials, W)
